# baseline (speedup 1.0000x reference)
import functools

import jax
import jax.numpy as jnp
from jax import lax
from jax.experimental import pallas as pl
from jax.experimental.pallas import tpu as pltpu
from jax.experimental.pallas import tpu_sc as plsc

N = 10000
NP = 10240
H = 128
E = 640000
NC = 2
NS = 16
NW = NC * NS
EW = E // NW
CH = 50
NCH = EW // CH
IB = 4
NB = NCH // IB
DCH = 125
DNCH = EW // DCH
RPT = NP // NS

_mesh = plsc.VectorSubcoreMesh(core_axis_name="c", subcore_axis_name="s",
                               num_cores=NC, num_subcores=NS)


def _fill_zero(ref, nrows):
    z = jnp.zeros((16,), jnp.float32)
    for i in range(nrows):
        for j in range(H // 16):
            ref[i, pl.ds(j * 16, 16)] = z


def _make_deg_kernel():
    @functools.partial(
        pl.kernel,
        out_type=jax.ShapeDtypeStruct((NC, NP), jnp.float32),
        mesh=_mesh,
        scratch_types=[
            pltpu.VMEM((DNCH, DCH), jnp.int32),
            pltpu.VMEM((RPT,), jnp.float32),
            pltpu.VMEM((128,), jnp.float32),
            pltpu.VMEM_SHARED((NP,), jnp.float32),
            pltpu.SemaphoreType.DMA,
            pltpu.SemaphoreType.DMA,
        ],
    )
    def deg_kernel(dst_hbm, out_hbm, didx, zbuf, ones, dacc, s0, s1):
        c = lax.axis_index("c")
        s = lax.axis_index("s")
        wid = c * NS + s
        zv = jnp.zeros((16,), jnp.float32)
        ov = jnp.ones((16,), jnp.float32)
        for i in range(RPT // 16):
            zbuf[pl.ds(i * 16, 16)] = zv
        for i in range(8):
            ones[pl.ds(i * 16, 16)] = ov
        pltpu.sync_copy(zbuf, dacc.at[pl.ds(s * RPT, RPT)])
        plsc.subcore_barrier()
        pltpu.sync_copy(dst_hbm.at[wid], didx)
        ones_c = ones.at[pl.ds(0, DCH)]
        pltpu.async_copy(ones_c, dacc.at[didx.at[0]], s0, add=True)

        def body(i, _):
            j = 2 * i
            pltpu.async_copy(ones_c, dacc.at[didx.at[j + 1]], s1, add=True)
            pltpu.make_async_copy(ones_c, dacc.at[didx.at[j]], s0).wait()

            @pl.when(j + 2 < DNCH)
            def _():
                pltpu.async_copy(ones_c, dacc.at[didx.at[j + 2]], s0,
                                 add=True)
            pltpu.make_async_copy(ones_c, dacc.at[didx.at[j + 1]], s1).wait()
            return 0

        lax.fori_loop(0, DNCH // 2, body, 0)
        plsc.subcore_barrier()
        pltpu.sync_copy(dacc.at[pl.ds(s * RPT, RPT)],
                        out_hbm.at[c, pl.ds(s * RPT, RPT)])

    return deg_kernel


_deg_kernel = _make_deg_kernel()


def _make_agg_kernel():
    @functools.partial(
        pl.kernel,
        out_type=jax.ShapeDtypeStruct((NC, NP, H), jnp.float32),
        mesh=_mesh,
        scratch_types=[
            pltpu.VMEM((4, IB, CH), jnp.int32),
            pltpu.VMEM((4, IB, CH), jnp.int32),
            pltpu.VMEM((4, CH, H), jnp.float32),
            pltpu.VMEM_SHARED((NP, H), jnp.float32),
            pltpu.SemaphoreType.DMA,
            pltpu.SemaphoreType.DMA,
            pltpu.SemaphoreType.DMA,
            pltpu.SemaphoreType.DMA,
            pltpu.SemaphoreType.DMA,
            pltpu.SemaphoreType.DMA,
            pltpu.SemaphoreType.DMA,
            pltpu.SemaphoreType.DMA,
            pltpu.SemaphoreType.DMA,
            pltpu.SemaphoreType.DMA,
            pltpu.SemaphoreType.DMA,
            pltpu.SemaphoreType.DMA,
            pltpu.SemaphoreType.DMA,
            pltpu.SemaphoreType.DMA,
            pltpu.SemaphoreType.DMA,
            pltpu.SemaphoreType.DMA,
        ],
    )
    def agg_kernel(src_hbm, dst_hbm, y_hbm, out_hbm,
                   sidx, didx, rows, acc,
                   g0, g1, g2, g3, s0, s1, s2, s3,
                   is0, is1, is2, is3, id0, id1, id2, id3):
        c = lax.axis_index("c")
        s = lax.axis_index("s")
        wid = c * NS + s
        gsems = (g0, g1, g2, g3)
        ssems = (s0, s1, s2, s3)
        isems = (is0, is1, is2, is3)
        dsems = (id0, id1, id2, id3)
        _fill_zero(rows.at[0], CH)
        base = s * RPT
        for k in range(RPT // CH):
            pltpu.sync_copy(rows.at[0], acc.at[pl.ds(base + k * CH, CH)])
        rem = RPT - (RPT // CH) * CH
        if rem:
            pltpu.sync_copy(rows.at[0, pl.ds(0, rem)],
                            acc.at[pl.ds(base + RPT - rem, rem)])
        plsc.subcore_barrier()
        pltpu.sync_copy(src_hbm.at[wid * NB], sidx.at[0])
        pltpu.sync_copy(dst_hbm.at[wid * NB], didx.at[0])
        pltpu.sync_copy(src_hbm.at[wid * NB + 1], sidx.at[1])
        pltpu.sync_copy(dst_hbm.at[wid * NB + 1], didx.at[1])
        pltpu.async_copy(src_hbm.at[wid * NB + 2], sidx.at[2], is2)
        pltpu.async_copy(dst_hbm.at[wid * NB + 2], didx.at[2], id2)
        pltpu.async_copy(y_hbm.at[sidx.at[0, 0]], rows.at[0], g0)
        pltpu.async_copy(y_hbm.at[sidx.at[0, 1]], rows.at[1], g1)

        def body(i4, _):
            for p in (0, 1, 2, 3):
                i = i4 * 4 + p
                q = (p + 1) % 4

                @pl.when(jnp.logical_and(i + 1 >= 2, i + 1 < NB))
                def _():
                    pltpu.make_async_copy(src_hbm.at[wid * NB + i + 1],
                                          sidx.at[q], isems[q]).wait()
                    pltpu.make_async_copy(dst_hbm.at[wid * NB + i + 1],
                                          didx.at[q], dsems[q]).wait()

                for b2 in range(IB):
                    j = i * IB + b2
                    b = b2 % 4
                    pltpu.make_async_copy(y_hbm.at[sidx.at[p, b2]],
                                          rows.at[b], gsems[b]).wait()
                    pltpu.async_copy(rows.at[b], acc.at[didx.at[p, b2]],
                                     ssems[b], add=True)
                    wb = (b2 - 2) % 4
                    if b2 >= 2:
                        widx = didx.at[p, b2 - 2]
                    else:
                        widx = didx.at[(p + 3) % 4, b2 + 2]

                    @pl.when(j >= 2)
                    def _():
                        pltpu.make_async_copy(rows.at[wb], acc.at[widx],
                                              ssems[wb]).wait()

                    gb = (b2 + 2) % 4
                    if b2 < IB - 2:
                        nidx = sidx.at[p, b2 + 2]
                    else:
                        nidx = sidx.at[q, b2 + 2 - IB]

                    @pl.when(j + 2 < NCH)
                    def _():
                        pltpu.async_copy(y_hbm.at[nidx], rows.at[gb],
                                         gsems[gb])

                r = (p + 3) % 4

                @pl.when(i + 3 < NB)
                def _():
                    pltpu.async_copy(src_hbm.at[wid * NB + i + 3], sidx.at[r],
                                     isems[r])
                    pltpu.async_copy(dst_hbm.at[wid * NB + i + 3], didx.at[r],
                                     dsems[r])
            return 0

        lax.fori_loop(0, NB // 4, body, 0)
        pltpu.make_async_copy(rows.at[2], acc.at[didx.at[3, IB - 2]],
                              ssems[2]).wait()
        pltpu.make_async_copy(rows.at[3], acc.at[didx.at[3, IB - 1]],
                              ssems[3]).wait()
        plsc.subcore_barrier()
        pltpu.sync_copy(acc.at[pl.ds(s * RPT, RPT)],
                        out_hbm.at[c, pl.ds(s * RPT, RPT)])

    return agg_kernel


_agg_kernel = _make_agg_kernel()


def _dot(a, b):
    return lax.dot_general(a, b, (((1,), (0,)), ((), ())),
                           preferred_element_type=jnp.float32)


def _tc1_body(x_ref, w_ref, deg_ref, y_ref, dinv_ref):
    d = deg_ref[...]
    dsum = (d[0] + d[1])[:N] + 1.0
    dinv = lax.rsqrt(dsum)
    dinv_ref[...] = dinv
    y_ref[...] = _dot(x_ref[...], w_ref[...]) * dinv


def _tc_mid_body(z_ref, y_ref, dinv_ref, b_ref, g_ref, be_ref, w_ref,
                 x_ref, ynext_ref):
    z = z_ref[...]
    dinv = dinv_ref[...]
    conv = (z[0, :N] + z[1, :N] + y_ref[...]) * dinv + b_ref[...]
    m = jnp.mean(conv, axis=0)
    v = jnp.mean((conv - m) ** 2, axis=0)
    xh = jnp.maximum((conv - m) * lax.rsqrt(v + 1e-5) * g_ref[...]
                     + be_ref[...], 0.0)
    x_ref[...] = xh
    ynext_ref[...] = _dot(xh, w_ref[...]) * dinv


def _tc_final_body(z_ref, y_ref, dinv_ref, b_ref, g_ref, be_ref,
                   x1_ref, x2_ref, wl1_ref, bl1_ref, wl2_ref, bl2_ref,
                   out_ref):
    z = z_ref[...]
    dinv = dinv_ref[...]
    conv = (z[0, :N] + z[1, :N] + y_ref[...]) * dinv + b_ref[...]
    m = jnp.mean(conv, axis=0)
    v = jnp.mean((conv - m) ** 2, axis=0)
    x3 = jnp.maximum((conv - m) * lax.rsqrt(v + 1e-5) * g_ref[...]
                     + be_ref[...], 0.0)
    h = x1_ref[...] + x2_ref[...] + x3
    t = jnp.maximum(_dot(h, wl1_ref[...]) + bl1_ref[...], 0.0)
    out_ref[...] = _dot(t, wl2_ref[...]) + bl2_ref[...]


_FS = jax.ShapeDtypeStruct
_TC_PARAMS = pltpu.CompilerParams(vmem_limit_bytes=100 * 1024 * 1024)

_tc1 = pl.pallas_call(
    _tc1_body,
    out_shape=(_FS((N, H), jnp.float32), _FS((N, 1), jnp.float32)),
    compiler_params=_TC_PARAMS,
)

_tc_mid = pl.pallas_call(
    _tc_mid_body,
    out_shape=(_FS((N, H), jnp.float32), _FS((N, H), jnp.float32)),
    compiler_params=_TC_PARAMS,
)

_tc_final = pl.pallas_call(
    _tc_final_body,
    out_shape=_FS((N, 1), jnp.float32),
    compiler_params=_TC_PARAMS,
)


def kernel(x, edge_index, W1, b1, W2, b2, W3, b3, g1, be1, g2, be2,
           g3, be3, Wl1, bl1, Wl2, bl2):
    src = edge_index[0].reshape(NW * NB, IB, CH)
    dst = edge_index[1].reshape(NW * NB, IB, CH)
    dst3 = edge_index[1].reshape(NW, DNCH, DCH)

    deg = _deg_kernel(dst3).reshape(NC, NP, 1)
    y1, dinv = _tc1(x, W1, deg)
    z1 = _agg_kernel(src, dst, y1)
    x1, y2 = _tc_mid(z1, y1, dinv, b1, g1, be1, W2)
    z2 = _agg_kernel(src, dst, y2)
    x2, y3 = _tc_mid(z2, y2, dinv, b2, g2, be2, W3)
    z3 = _agg_kernel(src, dst, y3)
    out = _tc_final(z3, y3, dinv, b3, g3, be3, x1, x2, Wl1, bl1, Wl2, bl2)
    return out.squeeze(-1)


# speedup vs baseline: 1.0116x; 1.0116x over previous
import functools

import jax
import jax.numpy as jnp
from jax import lax
from jax.experimental import pallas as pl
from jax.experimental.pallas import tpu as pltpu
from jax.experimental.pallas import tpu_sc as plsc

N = 10000
NP = 10240
H = 128
E = 640000
NC = 2
NS = 16
NW = NC * NS
EW = E // NW
CH = 125
NCH = EW // CH
IB = 10
NB = NCH // IB
RPT = NP // NS

_mesh = plsc.VectorSubcoreMesh(core_axis_name="c", subcore_axis_name="s",
                               num_cores=NC, num_subcores=NS)


def _fill_zero(ref, nrows):
    z = jnp.zeros((16,), jnp.float32)
    for i in range(nrows):
        for j in range(H // 16):
            ref[i, pl.ds(j * 16, 16)] = z


def _make_deg_kernel():
    @functools.partial(
        pl.kernel,
        out_type=jax.ShapeDtypeStruct((NC, NP), jnp.float32),
        mesh=_mesh,
        scratch_types=[
            pltpu.VMEM((NCH, CH), jnp.int32),
            pltpu.VMEM((RPT,), jnp.float32),
            pltpu.VMEM((128,), jnp.float32),
            pltpu.VMEM_SHARED((NP,), jnp.float32),
            pltpu.SemaphoreType.DMA,
            pltpu.SemaphoreType.DMA,
        ],
    )
    def deg_kernel(dst_hbm, out_hbm, didx, zbuf, ones, dacc, s0, s1):
        c = lax.axis_index("c")
        s = lax.axis_index("s")
        wid = c * NS + s
        zv = jnp.zeros((16,), jnp.float32)
        ov = jnp.ones((16,), jnp.float32)
        for i in range(RPT // 16):
            zbuf[pl.ds(i * 16, 16)] = zv
        for i in range(8):
            ones[pl.ds(i * 16, 16)] = ov
        pltpu.sync_copy(zbuf, dacc.at[pl.ds(s * RPT, RPT)])
        plsc.subcore_barrier()
        pltpu.sync_copy(dst_hbm.at[wid], didx)
        ones_c = ones.at[pl.ds(0, CH)]
        sems = (s0, s1)
        pltpu.async_copy(ones_c, dacc.at[didx.at[0]], s0, add=True)

        def body(i, _):
            j = 2 * i
            pltpu.async_copy(ones_c, dacc.at[didx.at[j + 1]], s1, add=True)
            pltpu.make_async_copy(ones_c, dacc.at[didx.at[j]], s0).wait()

            @pl.when(j + 2 < NCH)
            def _():
                pltpu.async_copy(ones_c, dacc.at[didx.at[j + 2]], s0,
                                 add=True)
            pltpu.make_async_copy(ones_c, dacc.at[didx.at[j + 1]], s1).wait()
            return 0

        lax.fori_loop(0, NCH // 2, body, 0)
        plsc.subcore_barrier()
        pltpu.sync_copy(dacc.at[pl.ds(s * RPT, RPT)],
                        out_hbm.at[c, pl.ds(s * RPT, RPT)])

    return deg_kernel


_deg_kernel = _make_deg_kernel()


def _make_agg_kernel():
    @functools.partial(
        pl.kernel,
        out_type=jax.ShapeDtypeStruct((NC, NP, H), jnp.float32),
        mesh=_mesh,
        scratch_types=[
            pltpu.VMEM((4, IB, CH), jnp.int32),
            pltpu.VMEM((4, IB, CH), jnp.int32),
            pltpu.VMEM((2, CH, H), jnp.float32),
            pltpu.VMEM_SHARED((NP, H), jnp.float32),
            pltpu.SemaphoreType.DMA,
            pltpu.SemaphoreType.DMA,
            pltpu.SemaphoreType.DMA,
            pltpu.SemaphoreType.DMA,
            pltpu.SemaphoreType.DMA,
            pltpu.SemaphoreType.DMA,
            pltpu.SemaphoreType.DMA,
            pltpu.SemaphoreType.DMA,
            pltpu.SemaphoreType.DMA,
            pltpu.SemaphoreType.DMA,
            pltpu.SemaphoreType.DMA,
            pltpu.SemaphoreType.DMA,
        ],
    )
    def agg_kernel(src_hbm, dst_hbm, y_hbm, out_hbm,
                   sidx, didx, rows, acc,
                   g0, g1, s0, s1, is0, is1, is2, is3, id0, id1,
                   id2, id3):
        c = lax.axis_index("c")
        s = lax.axis_index("s")
        wid = c * NS + s
        gsems = (g0, g1)
        ssems = (s0, s1)
        isems = (is0, is1, is2, is3)
        dsems = (id0, id1, id2, id3)
        _fill_zero(rows.at[0], CH)
        base = s * RPT
        for k in range(RPT // CH):
            pltpu.sync_copy(rows.at[0], acc.at[pl.ds(base + k * CH, CH)])
        rem = RPT - (RPT // CH) * CH
        if rem:
            pltpu.sync_copy(rows.at[0, pl.ds(0, rem)],
                            acc.at[pl.ds(base + RPT - rem, rem)])
        plsc.subcore_barrier()
        pltpu.sync_copy(src_hbm.at[wid * NB], sidx.at[0])
        pltpu.sync_copy(dst_hbm.at[wid * NB], didx.at[0])
        pltpu.sync_copy(src_hbm.at[wid * NB + 1], sidx.at[1])
        pltpu.sync_copy(dst_hbm.at[wid * NB + 1], didx.at[1])
        pltpu.async_copy(src_hbm.at[wid * NB + 2], sidx.at[2], is2)
        pltpu.async_copy(dst_hbm.at[wid * NB + 2], didx.at[2], id2)
        pltpu.async_copy(y_hbm.at[sidx.at[0, 0]], rows.at[0], g0)

        def body(i4, _):
            for p in (0, 1, 2, 3):
                i = i4 * 4 + p
                q = (p + 1) % 4

                @pl.when(jnp.logical_and(i + 1 >= 2, i + 1 < NB))
                def _():
                    pltpu.make_async_copy(src_hbm.at[wid * NB + i + 1],
                                          sidx.at[q], isems[q]).wait()
                    pltpu.make_async_copy(dst_hbm.at[wid * NB + i + 1],
                                          didx.at[q], dsems[q]).wait()

                for b2 in range(IB):
                    j = i * IB + b2
                    b = b2 % 2
                    pltpu.make_async_copy(y_hbm.at[sidx.at[p, b2]],
                                          rows.at[b], gsems[b]).wait()
                    pltpu.async_copy(rows.at[b], acc.at[didx.at[p, b2]],
                                     ssems[b], add=True)
                    if b2 >= 1:
                        widx = didx.at[p, b2 - 1]
                    else:
                        widx = didx.at[(p + 3) % 4, IB - 1]

                    @pl.when(j >= 1)
                    def _():
                        pltpu.make_async_copy(rows.at[1 - b], acc.at[widx],
                                              ssems[1 - b]).wait()

                    if b2 < IB - 1:
                        nidx = sidx.at[p, b2 + 1]
                    else:
                        nidx = sidx.at[q, 0]

                    @pl.when(j + 1 < NCH)
                    def _():
                        pltpu.async_copy(y_hbm.at[nidx], rows.at[1 - b],
                                         gsems[1 - b])

                r = (p + 3) % 4

                @pl.when(i + 3 < NB)
                def _():
                    pltpu.async_copy(src_hbm.at[wid * NB + i + 3], sidx.at[r],
                                     isems[r])
                    pltpu.async_copy(dst_hbm.at[wid * NB + i + 3], didx.at[r],
                                     dsems[r])
            return 0

        lax.fori_loop(0, NB // 4, body, 0)
        pltpu.make_async_copy(rows.at[1], acc.at[didx.at[3, IB - 1]],
                              ssems[1]).wait()
        plsc.subcore_barrier()
        pltpu.sync_copy(acc.at[pl.ds(s * RPT, RPT)],
                        out_hbm.at[c, pl.ds(s * RPT, RPT)])

    return agg_kernel


_agg_kernel = _make_agg_kernel()


def _dot(a, b):
    return lax.dot_general(a, b, (((1,), (0,)), ((), ())),
                           preferred_element_type=jnp.float32)


def _tc1_body(x_ref, w_ref, deg_ref, y_ref, dinv_ref):
    d = deg_ref[...]
    dsum = (d[0] + d[1])[:N] + 1.0
    dinv = lax.rsqrt(dsum)
    dinv_ref[...] = dinv
    y_ref[...] = _dot(x_ref[...], w_ref[...]) * dinv


def _tc_mid_body(z_ref, y_ref, dinv_ref, b_ref, g_ref, be_ref, w_ref,
                 x_ref, ynext_ref):
    z = z_ref[...]
    dinv = dinv_ref[...]
    conv = (z[0, :N] + z[1, :N] + y_ref[...]) * dinv + b_ref[...]
    m = jnp.mean(conv, axis=0)
    v = jnp.mean((conv - m) ** 2, axis=0)
    xh = jnp.maximum((conv - m) * lax.rsqrt(v + 1e-5) * g_ref[...]
                     + be_ref[...], 0.0)
    x_ref[...] = xh
    ynext_ref[...] = _dot(xh, w_ref[...]) * dinv


def _tc_final_body(z_ref, y_ref, dinv_ref, b_ref, g_ref, be_ref,
                   x1_ref, x2_ref, wl1_ref, bl1_ref, wl2_ref, bl2_ref,
                   out_ref):
    z = z_ref[...]
    dinv = dinv_ref[...]
    conv = (z[0, :N] + z[1, :N] + y_ref[...]) * dinv + b_ref[...]
    m = jnp.mean(conv, axis=0)
    v = jnp.mean((conv - m) ** 2, axis=0)
    x3 = jnp.maximum((conv - m) * lax.rsqrt(v + 1e-5) * g_ref[...]
                     + be_ref[...], 0.0)
    h = x1_ref[...] + x2_ref[...] + x3
    t = jnp.maximum(_dot(h, wl1_ref[...]) + bl1_ref[...], 0.0)
    out_ref[...] = _dot(t, wl2_ref[...]) + bl2_ref[...]


_FS = jax.ShapeDtypeStruct
_TC_PARAMS = pltpu.CompilerParams(vmem_limit_bytes=100 * 1024 * 1024)

_tc1 = pl.pallas_call(
    _tc1_body,
    out_shape=(_FS((N, H), jnp.float32), _FS((N, 1), jnp.float32)),
    compiler_params=_TC_PARAMS,
)

_tc_mid = pl.pallas_call(
    _tc_mid_body,
    out_shape=(_FS((N, H), jnp.float32), _FS((N, H), jnp.float32)),
    compiler_params=_TC_PARAMS,
)

_tc_final = pl.pallas_call(
    _tc_final_body,
    out_shape=_FS((N, 1), jnp.float32),
    compiler_params=_TC_PARAMS,
)


def kernel(x, edge_index, W1, b1, W2, b2, W3, b3, g1, be1, g2, be2,
           g3, be3, Wl1, bl1, Wl2, bl2):
    src = edge_index[0].reshape(NW * NB, IB, CH)
    dst = edge_index[1].reshape(NW * NB, IB, CH)
    dst3 = edge_index[1].reshape(NW, NCH, CH)

    deg = _deg_kernel(dst3).reshape(NC, NP, 1)
    y1, dinv = _tc1(x, W1, deg)
    z1 = _agg_kernel(src, dst, y1)
    x1, y2 = _tc_mid(z1, y1, dinv, b1, g1, be1, W2)
    z2 = _agg_kernel(src, dst, y2)
    x2, y3 = _tc_mid(z2, y2, dinv, b2, g2, be2, W3)
    z3 = _agg_kernel(src, dst, y3)
    out = _tc_final(z3, y3, dinv, b3, g3, be3, x1, x2, Wl1, bl1, Wl2, bl2)
    return out.squeeze(-1)


# speedup vs baseline: 1.2050x; 1.1912x over previous
import functools

import jax
import jax.numpy as jnp
from jax import lax
from jax.experimental import pallas as pl
from jax.experimental.pallas import tpu as pltpu
from jax.experimental.pallas import tpu_sc as plsc

N = 10000
NP = 10240
H = 128
E = 640000
NC = 2
NS = 16
NW = NC * NS
EW = E // NW
CH = 125
NCH = EW // CH
IB = 10
NB = NCH // IB
RPT = NP // NS

_mesh = plsc.VectorSubcoreMesh(core_axis_name="c", subcore_axis_name="s",
                               num_cores=NC, num_subcores=NS)


def _fill_zero(ref, nrows):
    z = jnp.zeros((16,), jnp.float32)
    for i in range(nrows):
        for j in range(H // 16):
            ref[i, pl.ds(j * 16, 16)] = z


def _make_deg_kernel():
    @functools.partial(
        pl.kernel,
        out_type=jax.ShapeDtypeStruct((NC, NP), jnp.float32),
        mesh=_mesh,
        scratch_types=[
            pltpu.VMEM((NCH, CH), jnp.int32),
            pltpu.VMEM((RPT,), jnp.float32),
            pltpu.VMEM((128,), jnp.float32),
            pltpu.VMEM_SHARED((NP,), jnp.float32),
            pltpu.SemaphoreType.DMA,
            pltpu.SemaphoreType.DMA,
        ],
    )
    def deg_kernel(dst_hbm, out_hbm, didx, zbuf, ones, dacc, s0, s1):
        c = lax.axis_index("c")
        s = lax.axis_index("s")
        wid = c * NS + s
        zv = jnp.zeros((16,), jnp.float32)
        ov = jnp.ones((16,), jnp.float32)
        for i in range(RPT // 16):
            zbuf[pl.ds(i * 16, 16)] = zv
        for i in range(8):
            ones[pl.ds(i * 16, 16)] = ov
        pltpu.sync_copy(zbuf, dacc.at[pl.ds(s * RPT, RPT)])
        plsc.subcore_barrier()
        pltpu.sync_copy(dst_hbm.at[wid], didx)
        ones_c = ones.at[pl.ds(0, CH)]
        sems = (s0, s1)
        pltpu.async_copy(ones_c, dacc.at[didx.at[0]], s0, add=True)

        def body(i, _):
            j = 2 * i
            pltpu.async_copy(ones_c, dacc.at[didx.at[j + 1]], s1, add=True)
            pltpu.make_async_copy(ones_c, dacc.at[didx.at[j]], s0).wait()

            @pl.when(j + 2 < NCH)
            def _():
                pltpu.async_copy(ones_c, dacc.at[didx.at[j + 2]], s0,
                                 add=True)
            pltpu.make_async_copy(ones_c, dacc.at[didx.at[j + 1]], s1).wait()
            return 0

        lax.fori_loop(0, NCH // 2, body, 0)
        plsc.subcore_barrier()
        pltpu.sync_copy(dacc.at[pl.ds(s * RPT, RPT)],
                        out_hbm.at[c, pl.ds(s * RPT, RPT)])

    return deg_kernel


_deg_kernel = _make_deg_kernel()


def _make_agg_kernel():
    @functools.partial(
        pl.kernel,
        out_type=jax.ShapeDtypeStruct((NC, NP, H), jnp.float32),
        mesh=_mesh,
        scratch_types=[
            pltpu.VMEM((4, IB, CH), jnp.int32),
            pltpu.VMEM((4, IB, CH), jnp.int32),
            pltpu.VMEM((2, CH, H), jnp.float32),
            pltpu.VMEM_SHARED((NP, H), jnp.float32),
            pltpu.SemaphoreType.DMA,
            pltpu.SemaphoreType.DMA,
            pltpu.SemaphoreType.DMA,
            pltpu.SemaphoreType.DMA,
            pltpu.SemaphoreType.DMA,
            pltpu.SemaphoreType.DMA,
            pltpu.SemaphoreType.DMA,
            pltpu.SemaphoreType.DMA,
            pltpu.SemaphoreType.DMA,
            pltpu.SemaphoreType.DMA,
        ],
    )
    def agg_kernel(src_hbm, dst_hbm, y_hbm, out_hbm,
                   sidx, didx, rows, acc,
                   g0, g1, is0, is1, is2, is3, id0, id1, id2, id3):
        c = lax.axis_index("c")
        s = lax.axis_index("s")
        wid = c * NS + s
        gsems = (g0, g1)
        isems = (is0, is1, is2, is3)
        dsems = (id0, id1, id2, id3)
        _fill_zero(rows.at[0], CH)
        base = s * RPT
        for k in range(RPT // CH):
            pltpu.sync_copy(rows.at[0], acc.at[pl.ds(base + k * CH, CH)])
        rem = RPT - (RPT // CH) * CH
        if rem:
            pltpu.sync_copy(rows.at[0, pl.ds(0, rem)],
                            acc.at[pl.ds(base + RPT - rem, rem)])
        plsc.subcore_barrier()
        pltpu.sync_copy(src_hbm.at[wid * NB], sidx.at[0])
        pltpu.sync_copy(dst_hbm.at[wid * NB], didx.at[0])
        pltpu.sync_copy(src_hbm.at[wid * NB + 1], sidx.at[1])
        pltpu.sync_copy(dst_hbm.at[wid * NB + 1], didx.at[1])
        pltpu.async_copy(src_hbm.at[wid * NB + 2], sidx.at[2], is2)
        pltpu.async_copy(dst_hbm.at[wid * NB + 2], didx.at[2], id2)
        pltpu.async_copy(y_hbm.at[sidx.at[0, 0]], rows.at[0], g0)
        pltpu.async_copy(y_hbm.at[sidx.at[0, 1]], rows.at[1], g1)

        def body(i4, _):
            for p in (0, 1, 2, 3):
                i = i4 * 4 + p
                q = (p + 1) % 4

                @pl.when(jnp.logical_and(i + 1 >= 2, i + 1 < NB))
                def _():
                    pltpu.make_async_copy(src_hbm.at[wid * NB + i + 1],
                                          sidx.at[q], isems[q]).wait()
                    pltpu.make_async_copy(dst_hbm.at[wid * NB + i + 1],
                                          didx.at[q], dsems[q]).wait()

                for b2 in range(IB):
                    j = i * IB + b2
                    b = b2 % 2
                    pltpu.make_async_copy(y_hbm.at[sidx.at[p, b2]],
                                          rows.at[b], gsems[b]).wait()
                    pltpu.sync_copy(rows.at[b], acc.at[didx.at[p, b2]],
                                    add=True)

                    @pl.when(j + 2 < NCH)
                    def _():
                        if b2 < IB - 2:
                            nidx = sidx.at[p, b2 + 2]
                        else:
                            nidx = sidx.at[q, b2 + 2 - IB]
                        pltpu.async_copy(y_hbm.at[nidx], rows.at[b],
                                         gsems[b])

                r = (p + 3) % 4

                @pl.when(i + 3 < NB)
                def _():
                    pltpu.async_copy(src_hbm.at[wid * NB + i + 3], sidx.at[r],
                                     isems[r])
                    pltpu.async_copy(dst_hbm.at[wid * NB + i + 3], didx.at[r],
                                     dsems[r])
            return 0

        lax.fori_loop(0, NB // 4, body, 0)
        plsc.subcore_barrier()
        pltpu.sync_copy(acc.at[pl.ds(s * RPT, RPT)],
                        out_hbm.at[c, pl.ds(s * RPT, RPT)])

    return agg_kernel


_agg_kernel = _make_agg_kernel()


def _dot(a, b):
    return lax.dot_general(a, b, (((1,), (0,)), ((), ())),
                           preferred_element_type=jnp.float32)


def _tc1_body(x_ref, w_ref, deg_ref, y_ref, dinv_ref):
    d = deg_ref[...]
    dsum = (d[0] + d[1])[:N] + 1.0
    dinv = lax.rsqrt(dsum)
    dinv_ref[...] = dinv
    y_ref[...] = _dot(x_ref[...], w_ref[...]) * dinv


def _tc_mid_body(z_ref, y_ref, dinv_ref, b_ref, g_ref, be_ref, w_ref,
                 x_ref, ynext_ref):
    z = z_ref[...]
    dinv = dinv_ref[...]
    conv = (z[0, :N] + z[1, :N] + y_ref[...]) * dinv + b_ref[...]
    m = jnp.mean(conv, axis=0)
    v = jnp.mean((conv - m) ** 2, axis=0)
    xh = jnp.maximum((conv - m) * lax.rsqrt(v + 1e-5) * g_ref[...]
                     + be_ref[...], 0.0)
    x_ref[...] = xh
    ynext_ref[...] = _dot(xh, w_ref[...]) * dinv


def _tc_final_body(z_ref, y_ref, dinv_ref, b_ref, g_ref, be_ref,
                   x1_ref, x2_ref, wl1_ref, bl1_ref, wl2_ref, bl2_ref,
                   out_ref):
    z = z_ref[...]
    dinv = dinv_ref[...]
    conv = (z[0, :N] + z[1, :N] + y_ref[...]) * dinv + b_ref[...]
    m = jnp.mean(conv, axis=0)
    v = jnp.mean((conv - m) ** 2, axis=0)
    x3 = jnp.maximum((conv - m) * lax.rsqrt(v + 1e-5) * g_ref[...]
                     + be_ref[...], 0.0)
    h = x1_ref[...] + x2_ref[...] + x3
    t = jnp.maximum(_dot(h, wl1_ref[...]) + bl1_ref[...], 0.0)
    out_ref[...] = _dot(t, wl2_ref[...]) + bl2_ref[...]


_FS = jax.ShapeDtypeStruct
_TC_PARAMS = pltpu.CompilerParams(vmem_limit_bytes=100 * 1024 * 1024)

_tc1 = pl.pallas_call(
    _tc1_body,
    out_shape=(_FS((N, H), jnp.float32), _FS((N, 1), jnp.float32)),
    compiler_params=_TC_PARAMS,
)

_tc_mid = pl.pallas_call(
    _tc_mid_body,
    out_shape=(_FS((N, H), jnp.float32), _FS((N, H), jnp.float32)),
    compiler_params=_TC_PARAMS,
)

_tc_final = pl.pallas_call(
    _tc_final_body,
    out_shape=_FS((N, 1), jnp.float32),
    compiler_params=_TC_PARAMS,
)


def kernel(x, edge_index, W1, b1, W2, b2, W3, b3, g1, be1, g2, be2,
           g3, be3, Wl1, bl1, Wl2, bl2):
    src = edge_index[0].reshape(NW * NB, IB, CH)
    dst = edge_index[1].reshape(NW * NB, IB, CH)
    dst3 = edge_index[1].reshape(NW, NCH, CH)

    deg = _deg_kernel(dst3).reshape(NC, NP, 1)
    y1, dinv = _tc1(x, W1, deg)
    z1 = _agg_kernel(src, dst, y1)
    x1, y2 = _tc_mid(z1, y1, dinv, b1, g1, be1, W2)
    z2 = _agg_kernel(src, dst, y2)
    x2, y3 = _tc_mid(z2, y2, dinv, b2, g2, be2, W3)
    z3 = _agg_kernel(src, dst, y3)
    out = _tc_final(z3, y3, dinv, b3, g3, be3, x1, x2, Wl1, bl1, Wl2, bl2)
    return out.squeeze(-1)
